# baseline (speedup 1.0000x reference)
import functools

import jax
import jax.numpy as jnp
from jax import lax
from jax.experimental import pallas as pl
from jax.experimental.pallas import tpu as pltpu
from jax.experimental.pallas import tpu_sc as plsc

_GRP = 128
_CHUNK = 128
_NBUF = 2
_Q = 4


@functools.lru_cache(maxsize=None)
def _build(N, F, num_cores, num_subcores):
    NW = num_cores * num_subcores
    PER_W = N // NW
    K = _CHUNK // _GRP
    NCHUNK = PER_W // _CHUNK
    IDX_ROWS = PER_W // _GRP
    assert NCHUNK >= 2 and NCHUNK % 2 == 0

    mesh = plsc.VectorSubcoreMesh(core_axis_name="c", subcore_axis_name="s")

    @functools.partial(
        pl.kernel,
        mesh=mesh,
        compiler_params=pltpu.CompilerParams(use_tc_tiling_on_sc=False),
        out_type=jax.ShapeDtypeStruct((N, F), jnp.float32),
        scratch_types=[
            pltpu.VMEM((IDX_ROWS, _GRP), jnp.int32),
            pltpu.VMEM((_NBUF * _CHUNK, F), jnp.float32),
            pltpu.SemaphoreType.DMA,
            pltpu.SemaphoreType.DMA,
            pltpu.SemaphoreType.DMA,
            pltpu.SemaphoreType.DMA,
        ],
    )
    def body(idx_hbm, table_hbm, out_hbm, idx_v, rows_v,
             sem_g0, sem_g1, sem_o0, sem_o1):
        wid = lax.axis_index("s") * num_cores + lax.axis_index("c")
        sem_g = (sem_g0, sem_g1)
        sem_o = (sem_o0, sem_o1)
        pltpu.sync_copy(idx_hbm.at[pl.ds(wid * IDX_ROWS, IDX_ROWS)], idx_v)

        def gathers(g, b):
            return [
                pltpu.make_async_copy(
                    table_hbm.at[idx_v.at[g * K + j]],
                    rows_v.at[pl.ds(b * _CHUNK + j * _GRP, _GRP)],
                    sem_g[b],
                )
                for j in range(K)
            ]

        def out_copy(g, b):
            return pltpu.make_async_copy(
                rows_v.at[pl.ds(b * _CHUNK, _CHUNK)],
                out_hbm.at[pl.ds(wid * PER_W + g * _CHUNK, _CHUNK)],
                sem_o[b],
            )

        for d in gathers(0, 0):
            d.start()
        for d in gathers(1, 1):
            d.start()
        for d in gathers(0, 0):
            d.wait()
        out_copy(0, 0).start()

        def main(go, carry):
            for off in range(2):
                g = 2 * go + 1 + off
                b = 1 - off
                out_copy(g - 1, 1 - b).wait()
                for d in gathers(g + 1, 1 - b):
                    d.start()
                for d in gathers(g, b):
                    d.wait()
                out_copy(g, b).start()
            return carry

        lax.fori_loop(0, (NCHUNK - 2) // 2, main, 0)

        out_copy(NCHUNK - 2, 0).wait()
        for d in gathers(NCHUNK - 1, 1):
            d.wait()
        out_copy(NCHUNK - 1, 1).start()
        out_copy(NCHUNK - 1, 1).wait()

    return body


def kernel(inputs, embedding):
    B, H = inputs.shape
    V, F = embedding.shape
    N = B * H
    info = plsc.get_sparse_core_info()
    idx = inputs.reshape(N // _GRP, _GRP).astype(jnp.int32)
    fn = _build(N // _Q, F, info.num_cores, info.num_subcores)
    rows_per_q = N // _Q // _GRP
    outs = [
        fn(lax.slice_in_dim(idx, q * rows_per_q, (q + 1) * rows_per_q),
           embedding)
        for q in range(_Q)
    ]
    return jnp.concatenate(outs, axis=0).reshape(B, H, F)


# speedup vs baseline: 1.2368x; 1.2368x over previous
import functools

import jax
import jax.numpy as jnp
from jax import lax
from jax.experimental import pallas as pl
from jax.experimental.pallas import tpu as pltpu
from jax.experimental.pallas import tpu_sc as plsc

_GRP = 128
_CHUNK = 512
_NBUF = 2


@functools.lru_cache(maxsize=None)
def _build(N, F, num_cores, num_subcores):
    NW = num_cores * num_subcores
    PER_W = N // NW
    K = _CHUNK // _GRP
    NCHUNK = PER_W // _CHUNK
    IDX_ROWS = PER_W // _GRP
    assert NCHUNK >= 2 and NCHUNK % 2 == 0

    mesh = plsc.VectorSubcoreMesh(core_axis_name="c", subcore_axis_name="s")

    @functools.partial(
        pl.kernel,
        mesh=mesh,
        compiler_params=pltpu.CompilerParams(use_tc_tiling_on_sc=False),
        out_type=jax.ShapeDtypeStruct((N, F), jnp.float32),
        scratch_types=[
            pltpu.VMEM((IDX_ROWS, _GRP), jnp.int32),
            pltpu.VMEM((_NBUF * _CHUNK, F), jnp.float32),
            pltpu.SemaphoreType.DMA,
            pltpu.SemaphoreType.DMA,
            pltpu.SemaphoreType.DMA,
            pltpu.SemaphoreType.DMA,
        ],
    )
    def body(idx_hbm, table_hbm, out_hbm, idx_v, rows_v,
             sem_g0, sem_g1, sem_o0, sem_o1):
        wid = lax.axis_index("s") * num_cores + lax.axis_index("c")
        sem_g = (sem_g0, sem_g1)
        sem_o = (sem_o0, sem_o1)
        pltpu.sync_copy(idx_hbm.at[pl.ds(wid * IDX_ROWS, IDX_ROWS)], idx_v)

        def gathers(g, b):
            return [
                pltpu.make_async_copy(
                    table_hbm.at[idx_v.at[g * K + j]],
                    rows_v.at[pl.ds(b * _CHUNK + j * _GRP, _GRP)],
                    sem_g[b],
                )
                for j in range(K)
            ]

        def out_copy(g, b):
            return pltpu.make_async_copy(
                rows_v.at[pl.ds(b * _CHUNK, _CHUNK)],
                out_hbm.at[pl.ds(wid * PER_W + g * _CHUNK, _CHUNK)],
                sem_o[b],
            )

        for d in gathers(0, 0):
            d.start()
        for d in gathers(1, 1):
            d.start()
        for d in gathers(0, 0):
            d.wait()
        out_copy(0, 0).start()

        def main(go, carry):
            for off in range(2):
                g = 2 * go + 1 + off
                b = 1 - off
                out_copy(g - 1, 1 - b).wait()
                for d in gathers(g + 1, 1 - b):
                    d.start()
                for d in gathers(g, b):
                    d.wait()
                out_copy(g, b).start()
            return carry

        lax.fori_loop(0, (NCHUNK - 2) // 2, main, 0)

        out_copy(NCHUNK - 2, 0).wait()
        for d in gathers(NCHUNK - 1, 1):
            d.wait()
        out_copy(NCHUNK - 1, 1).start()
        out_copy(NCHUNK - 1, 1).wait()

    return body


def kernel(inputs, embedding):
    B, H = inputs.shape
    V, F = embedding.shape
    N = B * H
    info = plsc.get_sparse_core_info()
    idx = inputs.reshape(N // _GRP, _GRP).astype(jnp.int32)
    out = _build(N, F, info.num_cores, info.num_subcores)(idx, embedding)
    return out.reshape(B, H, F)
